# baseline (speedup 1.0000x reference)
import functools

import jax
import jax.numpy as jnp
from jax import lax
from jax.experimental import pallas as pl
from jax.experimental.pallas import tpu as pltpu
from jax.experimental.pallas import tpu_sc as plsc

N_NODES = 10000
N_EDGES = 320000
D_FEAT = 128
UNITS = 128

NC = 2
NS = 16
NW = NC * NS
EW = N_EDGES // NW
C = 64
NIT = 13
NCHF = 12 * NIT
TAIL_E = EW - NCHF * C
BL = 6
BI = BL * C
EWP = NCHF * C + 2 * BI
R_TILE = 624
R_TAIL_BASE = NS * R_TILE
R_TAIL = N_NODES - R_TAIL_BASE
ZREM = R_TILE - (R_TILE // C) * C


def _sc_body(x_hbm, row_hbm, col_hbm, out_hbm, cnt_hbm,
             rbA, cbA, rbB, cbB, g0, g1, g2, g3, cnt_v, shared,
             semg0, semg1, semg2, semg3, sems0, sems1, sems2, sems3,
             semi_a, semi_b):
    g = (g0, g1, g2, g3)
    semg = (semg0, semg1, semg2, semg3)
    sems = (sems0, sems1, sems2, sems3)

    c = lax.axis_index("c")
    s = lax.axis_index("s")
    wid = c * NS + s
    base = wid * EWP

    pltpu.async_copy(row_hbm.at[pl.ds(base, BI)], rbA, semi_a)
    pltpu.async_copy(col_hbm.at[pl.ds(base, BI)], cbA, semi_a)

    def zrow(r, carry):
        for k in range(D_FEAT // 16):
            g3[r, pl.ds(k * 16, 16)] = jnp.zeros((16,), jnp.float32)
        return carry

    lax.fori_loop(0, C, zrow, 0)

    def zcnt(i, carry):
        cnt_v[pl.ds(i * 16, 16)] = jnp.zeros((16,), jnp.float32)
        return carry

    lax.fori_loop(0, N_NODES // 16, zcnt, 0)

    pltpu.make_async_copy(row_hbm.at[pl.ds(0, BI)], rbA, semi_a).wait()
    pltpu.make_async_copy(row_hbm.at[pl.ds(0, BI)], cbA, semi_a).wait()

    pltpu.async_copy(x_hbm.at[cbA.at[pl.ds(0, C)]], g0, semg0)
    pltpu.async_copy(x_hbm.at[cbA.at[pl.ds(C, C)]], g1, semg1)
    pltpu.async_copy(x_hbm.at[cbA.at[pl.ds(2 * C, C)]], g2, semg2)

    r0 = s * R_TILE
    for k in range(R_TILE // C):
        pltpu.async_copy(g3, shared.at[pl.ds(r0 + k * C, C)], sems0)
    pltpu.async_copy(g3.at[pl.ds(0, ZREM)],
                     shared.at[pl.ds(r0 + (R_TILE // C) * C, ZREM)], sems0)

    @pl.when(s == NS - 1)
    def _zero_tail():
        pltpu.sync_copy(g3.at[pl.ds(0, R_TAIL)],
                        shared.at[pl.ds(R_TAIL_BASE, R_TAIL)])

    for k in range(R_TILE // C):
        pltpu.make_async_copy(x_hbm.at[pl.ds(0, C)], g3, sems0).wait()
    pltpu.make_async_copy(x_hbm.at[pl.ds(0, ZREM)], g3.at[pl.ds(0, ZREM)],
                          sems0).wait()

    plsc.subcore_barrier()

    pltpu.async_copy(g3, shared.at[rbA.at[pl.ds(0, C)]], sems3, add=True)

    ones16 = jnp.ones((16,), jnp.float32)

    def wait_dma(buf, sem):
        pltpu.make_async_copy(x_hbm.at[pl.ds(0, C)], buf, sem).wait()

    def wait_idx(dst, sem):
        pltpu.make_async_copy(row_hbm.at[pl.ds(0, BI)], dst, sem).wait()

    def gather_issue(cb_buf, off, bp):
        pltpu.async_copy(x_hbm.at[cb_buf.at[pl.ds(off, C)]], g[bp], semg[bp])

    def scat_issue(rb_buf, off, bp):
        pltpu.async_copy(g[bp], shared.at[rb_buf.at[pl.ds(off, C)]],
                         sems[bp], add=True)

    def count_blk(rb_buf, off):
        for k in range(C // 16):
            idx = rb_buf[pl.ds(off + k * 16, 16)]
            plsc.addupdate_scatter(cnt_v, [idx], ones16)

    def body(t, carry):
        for p in range(12):
            bp = p % 4
            bnext = (p + 3) % 4
            wait_dma(g[bp], semg[bp])
            wait_dma(g[bnext], sems[bnext])
            if p == 0:
                off_b = base + t * (12 * C) + BI
                pltpu.async_copy(row_hbm.at[pl.ds(off_b, BI)], rbB, semi_b)
                pltpu.async_copy(col_hbm.at[pl.ds(off_b, BI)], cbB, semi_b)
            if p == 3:
                wait_idx(rbB, semi_b)
                wait_idx(cbB, semi_b)
            if p == 6:
                off_a = base + (t + 1) * (12 * C)
                pltpu.async_copy(row_hbm.at[pl.ds(off_a, BI)], rbA, semi_a)
                pltpu.async_copy(col_hbm.at[pl.ds(off_a, BI)], cbA, semi_a)
            if p == 9:
                wait_idx(rbA, semi_a)
                wait_idx(cbA, semi_a)
            l = p + 3
            if l <= 5:
                gather_issue(cbA, l * C, bnext)
            elif l <= 11:
                gather_issue(cbB, (l - 6) * C, bnext)
            else:
                gather_issue(cbA, (l - 12) * C, bnext)
            if p <= 5:
                rb_buf, off = rbA, p * C
            else:
                rb_buf, off = rbB, (p - 6) * C
            scat_issue(rb_buf, off, bp)
            count_blk(rb_buf, off)
        return carry

    lax.fori_loop(0, NIT, body, 0)

    wait_dma(g[0], semg[0])
    wait_dma(g[1], semg[1])
    wait_dma(g[2], semg[2])
    wait_dma(g[3], sems[3])

    pltpu.async_copy(x_hbm.at[cbA.at[pl.ds(0, TAIL_E)]],
                     g0.at[pl.ds(0, TAIL_E)], semg0)
    pltpu.make_async_copy(x_hbm.at[pl.ds(0, TAIL_E)],
                          g0.at[pl.ds(0, TAIL_E)], semg0).wait()
    pltpu.async_copy(g0.at[pl.ds(0, TAIL_E)],
                     shared.at[rbA.at[pl.ds(0, TAIL_E)]], sems0, add=True)
    idx_tail = rbA[pl.ds(0, 16)]
    plsc.addupdate_scatter(cnt_v, [idx_tail], ones16)
    pltpu.make_async_copy(x_hbm.at[pl.ds(0, TAIL_E)],
                          g0.at[pl.ds(0, TAIL_E)], sems0).wait()

    plsc.subcore_barrier()

    pltpu.async_copy(cnt_v, cnt_hbm.at[pl.ds(wid * N_NODES, N_NODES)], semg0)
    pltpu.sync_copy(
        shared.at[pl.ds(r0, R_TILE)],
        out_hbm.at[c, pl.ds(r0, R_TILE)],
    )

    @pl.when(s == NS - 1)
    def _out_tail():
        pltpu.sync_copy(
            shared.at[pl.ds(R_TAIL_BASE, R_TAIL)],
            out_hbm.at[c, pl.ds(R_TAIL_BASE, R_TAIL)],
        )

    pltpu.make_async_copy(
        cnt_v, cnt_hbm.at[pl.ds(wid * N_NODES, N_NODES)], semg0).wait()


_sc_aggregate = functools.partial(
    pl.kernel,
    out_type=(
        jax.ShapeDtypeStruct((NC, N_NODES, D_FEAT), jnp.float32),
        jax.ShapeDtypeStruct((NW * N_NODES,), jnp.float32),
    ),
    mesh=plsc.VectorSubcoreMesh(core_axis_name="c", subcore_axis_name="s"),
    compiler_params=pltpu.CompilerParams(needs_layout_passes=False),
    scratch_types=[
        pltpu.VMEM((BI,), jnp.int32),
        pltpu.VMEM((BI,), jnp.int32),
        pltpu.VMEM((BI,), jnp.int32),
        pltpu.VMEM((BI,), jnp.int32),
        pltpu.VMEM((C, D_FEAT), jnp.float32),
        pltpu.VMEM((C, D_FEAT), jnp.float32),
        pltpu.VMEM((C, D_FEAT), jnp.float32),
        pltpu.VMEM((C, D_FEAT), jnp.float32),
        pltpu.VMEM((N_NODES,), jnp.float32),
        pltpu.VMEM_SHARED((N_NODES, D_FEAT), jnp.float32),
        pltpu.SemaphoreType.DMA,
        pltpu.SemaphoreType.DMA,
        pltpu.SemaphoreType.DMA,
        pltpu.SemaphoreType.DMA,
        pltpu.SemaphoreType.DMA,
        pltpu.SemaphoreType.DMA,
        pltpu.SemaphoreType.DMA,
        pltpu.SemaphoreType.DMA,
        pltpu.SemaphoreType.DMA,
        pltpu.SemaphoreType.DMA,
    ],
)(_sc_body)


def _tc_body(part_ref, cnt_ref, x_ref, wn_ref, ws_ref, b_ref, out_ref):
    seg = part_ref[0] + part_ref[1]
    cnt = jnp.sum(cnt_ref[...], axis=1, keepdims=True)
    mean = seg / jnp.maximum(cnt, 1.0)
    h1 = jnp.dot(mean, wn_ref[...], preferred_element_type=jnp.float32)
    h2 = jnp.dot(x_ref[...], ws_ref[...], preferred_element_type=jnp.float32)
    h = jnp.concatenate([h1, h2], axis=1) + b_ref[...]
    out_ref[...] = jnp.maximum(h, 0.0)


_TC_BLOCK = 400


def _tc_epilogue(part, cnt_t, x, wn, ws, bias2d):
    grid = (N_NODES // _TC_BLOCK,)
    return pl.pallas_call(
        _tc_body,
        grid=grid,
        in_specs=[
            pl.BlockSpec((NC, _TC_BLOCK, D_FEAT), lambda i: (0, i, 0)),
            pl.BlockSpec((_TC_BLOCK, NW), lambda i: (i, 0)),
            pl.BlockSpec((_TC_BLOCK, D_FEAT), lambda i: (i, 0)),
            pl.BlockSpec((D_FEAT, UNITS), lambda i: (0, 0)),
            pl.BlockSpec((D_FEAT, UNITS), lambda i: (0, 0)),
            pl.BlockSpec((1, 2 * UNITS), lambda i: (0, 0)),
        ],
        out_specs=pl.BlockSpec((_TC_BLOCK, 2 * UNITS), lambda i: (i, 0)),
        out_shape=jax.ShapeDtypeStruct((N_NODES, 2 * UNITS), jnp.float32),
    )(part, cnt_t, x, wn, ws, bias2d)


def kernel(x, edge_index, neighs_kernel, self_kernel, bias):
    row = jnp.pad(edge_index[0].reshape(NW, EW),
                  ((0, 0), (0, EWP - EW))).reshape(-1)
    col = jnp.pad(edge_index[1].reshape(NW, EW),
                  ((0, 0), (0, EWP - EW))).reshape(-1)
    part, cnt = _sc_aggregate(x, row, col)
    cnt_t = cnt.reshape(NW, N_NODES).T
    return _tc_epilogue(part, cnt_t, x, neighs_kernel, self_kernel,
                        bias.reshape(1, 2 * UNITS))


# speedup vs baseline: 2.4247x; 2.4247x over previous
import functools

import jax
import jax.numpy as jnp
from jax import lax
from jax.experimental import pallas as pl
from jax.experimental.pallas import tpu as pltpu
from jax.experimental.pallas import tpu_sc as plsc

N_NODES = 10000
N_EDGES = 320000
D_FEAT = 128
UNITS = 128

NC = 2
NS = 16
NW = NC * NS
EW = N_EDGES // NW
C = 80
NCH = EW // C
BL = 6
BI = BL * C
NIT = 10
TAIL = NCH - 12 * NIT
EWP = EW + C
R_TILE = 624
R_TAIL_BASE = NS * R_TILE
R_TAIL = N_NODES - R_TAIL_BASE


def _sc_body(x_hbm, row_hbm, col_hbm, out_hbm, cnt_hbm,
             rbA, cbA, rbB, cbB, g0, g1, g2, cnt_v, shared,
             semg0, semg1, semg2, sems0, sems1, sems2, semi_a, semi_b):
    g = (g0, g1, g2)
    semg = (semg0, semg1, semg2)
    sems = (sems0, sems1, sems2)

    c = lax.axis_index("c")
    s = lax.axis_index("s")
    wid = c * NS + s
    base = wid * EWP

    pltpu.async_copy(row_hbm.at[pl.ds(base, BI)], rbA, semi_a)
    pltpu.async_copy(col_hbm.at[pl.ds(base, BI)], cbA, semi_a)

    def zrow(r, carry):
        for k in range(D_FEAT // 16):
            g2[r, pl.ds(k * 16, 16)] = jnp.zeros((16,), jnp.float32)
        return carry

    lax.fori_loop(0, C, zrow, 0)

    def zcnt(i, carry):
        cnt_v[pl.ds(i * 16, 16)] = jnp.zeros((16,), jnp.float32)
        return carry

    lax.fori_loop(0, N_NODES // 16, zcnt, 0)

    pltpu.make_async_copy(row_hbm.at[pl.ds(0, BI)], rbA, semi_a).wait()
    pltpu.make_async_copy(row_hbm.at[pl.ds(0, BI)], cbA, semi_a).wait()

    pltpu.async_copy(x_hbm.at[cbA.at[pl.ds(0, C)]], g0, semg0)
    pltpu.async_copy(x_hbm.at[cbA.at[pl.ds(C, C)]], g1, semg1)

    r0 = s * R_TILE
    ZREM = R_TILE - (R_TILE // C) * C
    for k in range(R_TILE // C):
        pltpu.async_copy(g2, shared.at[pl.ds(r0 + k * C, C)], sems0)
    pltpu.async_copy(g2.at[pl.ds(0, ZREM)],
                     shared.at[pl.ds(r0 + (R_TILE // C) * C, ZREM)], sems0)

    @pl.when(s == NS - 1)
    def _zero_tail():
        pltpu.sync_copy(g2.at[pl.ds(0, R_TAIL)],
                        shared.at[pl.ds(R_TAIL_BASE, R_TAIL)])

    for k in range(R_TILE // C):
        pltpu.make_async_copy(x_hbm.at[pl.ds(0, C)], g2, sems0).wait()
    pltpu.make_async_copy(x_hbm.at[pl.ds(0, ZREM)], g2.at[pl.ds(0, ZREM)],
                          sems0).wait()

    plsc.subcore_barrier()

    pltpu.async_copy(g2, shared.at[rbA.at[pl.ds(0, C)]], sems2, add=True)

    ones16 = jnp.ones((16,), jnp.float32)

    def wait_dma(buf, sem):
        pltpu.make_async_copy(x_hbm.at[pl.ds(0, C)], buf, sem).wait()

    def wait_idx(dst, sem):
        pltpu.make_async_copy(row_hbm.at[pl.ds(0, BI)], dst, sem).wait()

    def gather_issue(cb_buf, off, bp):
        pltpu.async_copy(x_hbm.at[cb_buf.at[pl.ds(off, C)]], g[bp], semg[bp])

    def scat_issue(rb_buf, off, bp):
        pltpu.async_copy(g[bp], shared.at[rb_buf.at[pl.ds(off, C)]],
                         sems[bp], add=True)

    def count_blk(rb_buf, off):
        for k in range(C // 16):
            idx = rb_buf[pl.ds(off + k * 16, 16)]
            plsc.addupdate_scatter(cnt_v, [idx], ones16)

    def body(t, carry):
        for p in range(12):
            bp = p % 3
            bprev = (p + 2) % 3
            wait_dma(g[bp], semg[bp])
            wait_dma(g[bprev], sems[bprev])
            if p == 0:
                off_b = base + t * (12 * C) + BI
                pltpu.async_copy(row_hbm.at[pl.ds(off_b, BI)], rbB, semi_b)
                pltpu.async_copy(col_hbm.at[pl.ds(off_b, BI)], cbB, semi_b)
            if p == 4:
                wait_idx(rbB, semi_b)
                wait_idx(cbB, semi_b)
            if p == 6:
                off_a = base + (t + 1) * (12 * C)
                pltpu.async_copy(row_hbm.at[pl.ds(off_a, BI)], rbA, semi_a)
                pltpu.async_copy(col_hbm.at[pl.ds(off_a, BI)], cbA, semi_a)
            if p == 10:
                wait_idx(rbA, semi_a)
                wait_idx(cbA, semi_a)
            jp2 = p + 2
            if jp2 <= 5:
                gather_issue(cbA, jp2 * C, bprev)
            elif jp2 <= 11:
                gather_issue(cbB, (jp2 - 6) * C, bprev)
            else:
                gather_issue(cbA, (jp2 - 12) * C, bprev)
            if p <= 5:
                rb_buf, off = rbA, p * C
            else:
                rb_buf, off = rbB, (p - 6) * C
            scat_issue(rb_buf, off, bp)
            count_blk(rb_buf, off)
        return carry

    lax.fori_loop(0, NIT, body, 0)

    for p in range(TAIL):
        bp = p % 3
        bprev = (p + 2) % 3
        wait_dma(g[bp], semg[bp])
        wait_dma(g[bprev], sems[bprev])
        jp2 = p + 2
        if jp2 < TAIL:
            gather_issue(cbA, jp2 * C, bprev)
        scat_issue(rbA, p * C, bp)
        count_blk(rbA, p * C)
    wait_dma(g[(TAIL - 1) % 3], sems[(TAIL - 1) % 3])

    plsc.subcore_barrier()

    pltpu.async_copy(cnt_v, cnt_hbm.at[pl.ds(wid * N_NODES, N_NODES)], semg0)
    pltpu.sync_copy(
        shared.at[pl.ds(r0, R_TILE)],
        out_hbm.at[c, pl.ds(r0, R_TILE)],
    )

    @pl.when(s == NS - 1)
    def _out_tail():
        pltpu.sync_copy(
            shared.at[pl.ds(R_TAIL_BASE, R_TAIL)],
            out_hbm.at[c, pl.ds(R_TAIL_BASE, R_TAIL)],
        )

    pltpu.make_async_copy(
        cnt_v, cnt_hbm.at[pl.ds(wid * N_NODES, N_NODES)], semg0).wait()


_sc_aggregate = functools.partial(
    pl.kernel,
    out_type=(
        jax.ShapeDtypeStruct((NC, N_NODES, D_FEAT), jnp.float32),
        jax.ShapeDtypeStruct((NW * N_NODES,), jnp.float32),
    ),
    mesh=plsc.VectorSubcoreMesh(core_axis_name="c", subcore_axis_name="s"),
    compiler_params=pltpu.CompilerParams(needs_layout_passes=False),
    scratch_types=[
        pltpu.VMEM((BI,), jnp.int32),
        pltpu.VMEM((BI,), jnp.int32),
        pltpu.VMEM((BI,), jnp.int32),
        pltpu.VMEM((BI,), jnp.int32),
        pltpu.VMEM((C, D_FEAT), jnp.float32),
        pltpu.VMEM((C, D_FEAT), jnp.float32),
        pltpu.VMEM((C, D_FEAT), jnp.float32),
        pltpu.VMEM((N_NODES,), jnp.float32),
        pltpu.VMEM_SHARED((N_NODES, D_FEAT), jnp.float32),
        pltpu.SemaphoreType.DMA,
        pltpu.SemaphoreType.DMA,
        pltpu.SemaphoreType.DMA,
        pltpu.SemaphoreType.DMA,
        pltpu.SemaphoreType.DMA,
        pltpu.SemaphoreType.DMA,
        pltpu.SemaphoreType.DMA,
        pltpu.SemaphoreType.DMA,
    ],
)(_sc_body)


def _tc_body(part_ref, cnt_ref, x_ref, wn_ref, ws_ref, b_ref, out_ref):
    seg = part_ref[0] + part_ref[1]
    cnt = jnp.sum(cnt_ref[...], axis=1, keepdims=True)
    mean = seg / jnp.maximum(cnt, 1.0)
    h1 = jnp.dot(mean, wn_ref[...], preferred_element_type=jnp.float32)
    h2 = jnp.dot(x_ref[...], ws_ref[...], preferred_element_type=jnp.float32)
    h = jnp.concatenate([h1, h2], axis=1) + b_ref[...]
    out_ref[...] = jnp.maximum(h, 0.0)


_TC_BLOCK = 1000


def _tc_epilogue(part, cnt_t, x, wn, ws, bias2d):
    grid = (N_NODES // _TC_BLOCK,)
    return pl.pallas_call(
        _tc_body,
        grid=grid,
        in_specs=[
            pl.BlockSpec((NC, _TC_BLOCK, D_FEAT), lambda i: (0, i, 0)),
            pl.BlockSpec((_TC_BLOCK, NW), lambda i: (i, 0)),
            pl.BlockSpec((_TC_BLOCK, D_FEAT), lambda i: (i, 0)),
            pl.BlockSpec((D_FEAT, UNITS), lambda i: (0, 0)),
            pl.BlockSpec((D_FEAT, UNITS), lambda i: (0, 0)),
            pl.BlockSpec((1, 2 * UNITS), lambda i: (0, 0)),
        ],
        out_specs=pl.BlockSpec((_TC_BLOCK, 2 * UNITS), lambda i: (i, 0)),
        out_shape=jax.ShapeDtypeStruct((N_NODES, 2 * UNITS), jnp.float32),
    )(part, cnt_t, x, wn, ws, bias2d)


def kernel(x, edge_index, neighs_kernel, self_kernel, bias):
    row = jnp.pad(edge_index[0].reshape(NW, EW),
                  ((0, 0), (0, EWP - EW))).reshape(-1)
    col = jnp.pad(edge_index[1].reshape(NW, EW),
                  ((0, 0), (0, EWP - EW))).reshape(-1)
    part, cnt = _sc_aggregate(x, row, col)
    cnt_t = cnt.reshape(NW, N_NODES).T
    return _tc_epilogue(part, cnt_t, x, neighs_kernel, self_kernel,
                        bias.reshape(1, 2 * UNITS))


# speedup vs baseline: 2.4626x; 1.0157x over previous
import functools

import jax
import jax.numpy as jnp
from jax import lax
from jax.experimental import pallas as pl
from jax.experimental.pallas import tpu as pltpu
from jax.experimental.pallas import tpu_sc as plsc

N_NODES = 10000
N_EDGES = 320000
D_FEAT = 128
UNITS = 128

NC = 2
NS = 16
NW = NC * NS
EW = N_EDGES // NW
C = 80
NCH = EW // C
BL = 6
BI = BL * C
NIT = 10
TAIL = NCH - 12 * NIT
EWP = EW + C
R_TILE = 624
R_TAIL_BASE = NS * R_TILE
R_TAIL = N_NODES - R_TAIL_BASE


def _sc_body(x_hbm, row_hbm, col_hbm, out_hbm, cnt_hbm,
             rbA, cbA, rbB, cbB, g0, g1, g2, cnt_v, shared,
             semg0, semg1, semg2, sems0, sems1, sems2, semi_a, semi_b):
    g = (g0, g1, g2)
    semg = (semg0, semg1, semg2)
    sems = (sems0, sems1, sems2)

    c = lax.axis_index("c")
    s = lax.axis_index("s")
    wid = c * NS + s
    base = wid * EWP

    pltpu.async_copy(row_hbm.at[pl.ds(base, BI)], rbA, semi_a)
    pltpu.async_copy(col_hbm.at[pl.ds(base, BI)], cbA, semi_a)

    def zrow(r, carry):
        for k in range(D_FEAT // 16):
            g2[r, pl.ds(k * 16, 16)] = jnp.zeros((16,), jnp.float32)
        return carry

    lax.fori_loop(0, C, zrow, 0)

    def zcnt(i, carry):
        cnt_v[pl.ds(i * 16, 16)] = jnp.zeros((16,), jnp.float32)
        return carry

    lax.fori_loop(0, N_NODES // 16, zcnt, 0)

    pltpu.make_async_copy(row_hbm.at[pl.ds(0, BI)], rbA, semi_a).wait()
    pltpu.make_async_copy(row_hbm.at[pl.ds(0, BI)], cbA, semi_a).wait()

    pltpu.async_copy(x_hbm.at[cbA.at[pl.ds(0, C)]], g0, semg0)
    pltpu.async_copy(x_hbm.at[cbA.at[pl.ds(C, C)]], g1, semg1)

    r0 = s * R_TILE
    ZREM = R_TILE - (R_TILE // C) * C
    for k in range(R_TILE // C):
        pltpu.async_copy(g2, shared.at[pl.ds(r0 + k * C, C)], sems0)
    pltpu.async_copy(g2.at[pl.ds(0, ZREM)],
                     shared.at[pl.ds(r0 + (R_TILE // C) * C, ZREM)], sems0)

    @pl.when(s == NS - 1)
    def _zero_tail():
        pltpu.sync_copy(g2.at[pl.ds(0, R_TAIL)],
                        shared.at[pl.ds(R_TAIL_BASE, R_TAIL)])

    for k in range(R_TILE // C):
        pltpu.make_async_copy(x_hbm.at[pl.ds(0, C)], g2, sems0).wait()
    pltpu.make_async_copy(x_hbm.at[pl.ds(0, ZREM)], g2.at[pl.ds(0, ZREM)],
                          sems0).wait()

    plsc.subcore_barrier()

    pltpu.async_copy(g2, shared.at[rbA.at[pl.ds(0, C)]], sems2, add=True)

    ones16 = jnp.ones((16,), jnp.float32)

    def wait_dma(buf, sem):
        pltpu.make_async_copy(x_hbm.at[pl.ds(0, C)], buf, sem).wait()

    def wait_idx(dst, sem):
        pltpu.make_async_copy(row_hbm.at[pl.ds(0, BI)], dst, sem).wait()

    def gather_issue(cb_buf, off, bp):
        pltpu.async_copy(x_hbm.at[cb_buf.at[pl.ds(off, C)]], g[bp], semg[bp])

    def scat_issue(rb_buf, off, bp):
        pltpu.async_copy(g[bp], shared.at[rb_buf.at[pl.ds(off, C)]],
                         sems[bp], add=True)

    def count_blk(rb_buf, off):
        for k in range(C // 16):
            idx = rb_buf[pl.ds(off + k * 16, 16)]
            plsc.addupdate_scatter(cnt_v, [idx], ones16)

    def body(t, carry):
        for p in range(12):
            bp = p % 3
            bprev = (p + 2) % 3
            wait_dma(g[bp], semg[bp])
            wait_dma(g[bprev], sems[bprev])
            if p == 0:
                off_b = base + t * (12 * C) + BI
                pltpu.async_copy(row_hbm.at[pl.ds(off_b, BI)], rbB, semi_b)
                pltpu.async_copy(col_hbm.at[pl.ds(off_b, BI)], cbB, semi_b)
            if p == 4:
                wait_idx(rbB, semi_b)
                wait_idx(cbB, semi_b)
            if p == 6:
                off_a = base + (t + 1) * (12 * C)
                pltpu.async_copy(row_hbm.at[pl.ds(off_a, BI)], rbA, semi_a)
                pltpu.async_copy(col_hbm.at[pl.ds(off_a, BI)], cbA, semi_a)
            if p == 10:
                wait_idx(rbA, semi_a)
                wait_idx(cbA, semi_a)
            jp2 = p + 2
            if jp2 <= 5:
                gather_issue(cbA, jp2 * C, bprev)
            elif jp2 <= 11:
                gather_issue(cbB, (jp2 - 6) * C, bprev)
            else:
                gather_issue(cbA, (jp2 - 12) * C, bprev)
            if p <= 5:
                rb_buf, off = rbA, p * C
            else:
                rb_buf, off = rbB, (p - 6) * C
            scat_issue(rb_buf, off, bp)
            count_blk(rb_buf, off)
        return carry

    lax.fori_loop(0, NIT, body, 0)

    for p in range(TAIL):
        bp = p % 3
        bprev = (p + 2) % 3
        wait_dma(g[bp], semg[bp])
        wait_dma(g[bprev], sems[bprev])
        jp2 = p + 2
        if jp2 < TAIL:
            gather_issue(cbA, jp2 * C, bprev)
        scat_issue(rbA, p * C, bp)
        count_blk(rbA, p * C)
    wait_dma(g[(TAIL - 1) % 3], sems[(TAIL - 1) % 3])

    plsc.subcore_barrier()

    pltpu.async_copy(cnt_v, cnt_hbm.at[pl.ds(wid * N_NODES, N_NODES)], semg0)
    pltpu.sync_copy(
        shared.at[pl.ds(r0, R_TILE)],
        out_hbm.at[c, pl.ds(r0, R_TILE)],
    )

    @pl.when(s == NS - 1)
    def _out_tail():
        pltpu.sync_copy(
            shared.at[pl.ds(R_TAIL_BASE, R_TAIL)],
            out_hbm.at[c, pl.ds(R_TAIL_BASE, R_TAIL)],
        )

    pltpu.make_async_copy(
        cnt_v, cnt_hbm.at[pl.ds(wid * N_NODES, N_NODES)], semg0).wait()


_sc_aggregate = functools.partial(
    pl.kernel,
    out_type=(
        jax.ShapeDtypeStruct((NC, N_NODES, D_FEAT), jnp.float32),
        jax.ShapeDtypeStruct((NW * N_NODES,), jnp.float32),
    ),
    mesh=plsc.VectorSubcoreMesh(core_axis_name="c", subcore_axis_name="s"),
    compiler_params=pltpu.CompilerParams(needs_layout_passes=False),
    scratch_types=[
        pltpu.VMEM((BI,), jnp.int32),
        pltpu.VMEM((BI,), jnp.int32),
        pltpu.VMEM((BI,), jnp.int32),
        pltpu.VMEM((BI,), jnp.int32),
        pltpu.VMEM((C, D_FEAT), jnp.float32),
        pltpu.VMEM((C, D_FEAT), jnp.float32),
        pltpu.VMEM((C, D_FEAT), jnp.float32),
        pltpu.VMEM((N_NODES,), jnp.float32),
        pltpu.VMEM_SHARED((N_NODES, D_FEAT), jnp.float32),
        pltpu.SemaphoreType.DMA,
        pltpu.SemaphoreType.DMA,
        pltpu.SemaphoreType.DMA,
        pltpu.SemaphoreType.DMA,
        pltpu.SemaphoreType.DMA,
        pltpu.SemaphoreType.DMA,
        pltpu.SemaphoreType.DMA,
        pltpu.SemaphoreType.DMA,
    ],
)(_sc_body)


def _tc_body(part_ref, cnt_ref, x_ref, wn_ref, ws_ref, b_ref, out_ref):
    seg = part_ref[0] + part_ref[1]
    cnt = jnp.sum(cnt_ref[...], axis=1, keepdims=True)
    mean = seg / jnp.maximum(cnt, 1.0)
    h1 = jnp.dot(mean, wn_ref[...], preferred_element_type=jnp.float32)
    h2 = jnp.dot(x_ref[...], ws_ref[...], preferred_element_type=jnp.float32)
    h = jnp.concatenate([h1, h2], axis=1) + b_ref[...]
    out_ref[...] = jnp.maximum(h, 0.0)


_TC_BLOCK = 2000


def _tc_epilogue(part, cnt_t, x, wn, ws, bias2d):
    grid = (N_NODES // _TC_BLOCK,)
    return pl.pallas_call(
        _tc_body,
        grid=grid,
        in_specs=[
            pl.BlockSpec((NC, _TC_BLOCK, D_FEAT), lambda i: (0, i, 0)),
            pl.BlockSpec((_TC_BLOCK, NW), lambda i: (i, 0)),
            pl.BlockSpec((_TC_BLOCK, D_FEAT), lambda i: (i, 0)),
            pl.BlockSpec((D_FEAT, UNITS), lambda i: (0, 0)),
            pl.BlockSpec((D_FEAT, UNITS), lambda i: (0, 0)),
            pl.BlockSpec((1, 2 * UNITS), lambda i: (0, 0)),
        ],
        out_specs=pl.BlockSpec((_TC_BLOCK, 2 * UNITS), lambda i: (i, 0)),
        out_shape=jax.ShapeDtypeStruct((N_NODES, 2 * UNITS), jnp.float32),
    )(part, cnt_t, x, wn, ws, bias2d)


def kernel(x, edge_index, neighs_kernel, self_kernel, bias):
    row = jnp.pad(edge_index[0].reshape(NW, EW),
                  ((0, 0), (0, EWP - EW))).reshape(-1)
    col = jnp.pad(edge_index[1].reshape(NW, EW),
                  ((0, 0), (0, EWP - EW))).reshape(-1)
    part, cnt = _sc_aggregate(x, row, col)
    cnt_t = cnt.reshape(NW, N_NODES).T
    return _tc_epilogue(part, cnt_t, x, neighs_kernel, self_kernel,
                        bias.reshape(1, 2 * UNITS))
